# baseline (speedup 1.0000x reference)
import jax
import jax.numpy as jnp
from jax.experimental import pallas as pl
from jax.experimental.pallas import tpu as pltpu

DIM = 128
M = 0.999
NUM_CLASSES = 1000
BUF = 16
DMLP = 2048
B = 128
NCP = 1024
F32 = jnp.float32


BB1 = 4


def _img_reduce_body(imq_ref, imk_ref, r_ref, c_ref, oq_ref, ok_ref):
    rm = r_ref[...]
    cm = c_ref[...]

    def red(x):
        xb = x.reshape(BB1 * 672, 224).astype(jnp.bfloat16)
        z = jnp.dot(rm, xb, preferred_element_type=F32)
        w = jnp.dot(z, cm, preferred_element_type=F32,
                    precision=jax.lax.Precision.HIGHEST)
        return (w * (1.0 / 196.0)).reshape(BB1, 48, 16)

    oq_ref[...] = red(imq_ref[...])
    ok_ref[...] = red(imk_ref[...])


def _img_reduce(imq, imk):
    ii = jax.lax.broadcasted_iota(jnp.int32, (48 * BB1, 672 * BB1), 0)
    jj = jax.lax.broadcasted_iota(jnp.int32, (48 * BB1, 672 * BB1), 1)
    rsel = ((ii // 48 == jj // 672) & ((ii % 48) // 16 == (jj % 672) // 224)
            & (ii % 16 == jj % 16)).astype(jnp.bfloat16)
    cj = jax.lax.broadcasted_iota(jnp.int32, (224, 16), 0)
    ck = jax.lax.broadcasted_iota(jnp.int32, (224, 16), 1)
    csel = (cj % 16 == ck).astype(F32)

    NCHUNK = 4
    CB = B // NCHUNK
    out_sh = jax.ShapeDtypeStruct((CB, 48, 16), F32)
    oqs, oks = [], []
    for c in range(NCHUNK):
        oq, ok = pl.pallas_call(
            _img_reduce_body,
            grid=(CB // BB1,),
            in_specs=[
                pl.BlockSpec((BB1, 3, 224, 224), lambda b: (b, 0, 0, 0)),
                pl.BlockSpec((BB1, 3, 224, 224), lambda b: (b, 0, 0, 0)),
                pl.BlockSpec((48 * BB1, 672 * BB1), lambda b: (0, 0)),
                pl.BlockSpec((224, 16), lambda b: (0, 0)),
            ],
            out_specs=[
                pl.BlockSpec((BB1, 48, 16), lambda b: (b, 0, 0)),
                pl.BlockSpec((BB1, 48, 16), lambda b: (b, 0, 0)),
            ],
            out_shape=[out_sh, out_sh],
        )(imq[c * CB:(c + 1) * CB], imk[c * CB:(c + 1) * CB], rsel, csel)
        oqs.append(oq)
        oks.append(ok)
    oq = jnp.concatenate(oqs, axis=0)
    ok = jnp.concatenate(oks, axis=0)
    return oq.reshape(B, 768), ok.reshape(B, 768)


def _proj_body(mq_ref, mk_ref, wpq_ref, wpk_ref, fq_ref, fk_ref):
    mq = mq_ref[...]
    mk = mk_ref[...]
    wpq = wpq_ref[...]
    wpqb = wpq.astype(jnp.bfloat16).astype(F32)
    wpkb = (M * wpk_ref[...] + (1.0 - M) * wpq).astype(jnp.bfloat16).astype(F32)
    hi = jax.lax.Precision.HIGHEST
    fq_ref[...] = jnp.maximum(jnp.dot(mq, wpqb, preferred_element_type=F32, precision=hi), 0.0)
    fk_ref[...] = jnp.maximum(jnp.dot(mk, wpkb, preferred_element_type=F32, precision=hi), 0.0)


def _proj(mq, mk, wpq, wpk):
    out_sh = jax.ShapeDtypeStruct((B, DMLP), F32)
    return pl.pallas_call(
        _proj_body,
        out_shape=[out_sh, out_sh],
    )(mq, mk, wpq, wpk)


def _logits_body(fq_ref, wlin_ref, blin_ref, tgt_ref, ep_ref, idx_ref, ptr_ref,
                 conf_ref, logits_ref, pred_ref, prob_ref, rw_ref, rr_ref,
                 live_ref, ptrn_ref, cg_ref, cn_ref):
    fqb = fq_ref[...].astype(jnp.bfloat16)
    wlb = wlin_ref[...].astype(jnp.bfloat16)
    lg = jnp.dot(fqb, wlb, preferred_element_type=F32) + blin_ref[...]
    logits_ref[...] = lg

    mx = jnp.max(lg, axis=1, keepdims=True)
    li = jax.lax.broadcasted_iota(jnp.int32, (B, NUM_CLASSES), 1)
    am = jnp.min(jnp.where(lg >= mx, li, NUM_CLASSES), axis=1)
    ep = ep_ref[0]
    tgt = tgt_ref[0]
    pred = jnp.where(ep < 0, tgt, am)
    pred_ref[0] = pred

    sig = 1.0 / (1.0 + jnp.exp(-lg))
    cls_iota = jax.lax.broadcasted_iota(jnp.int32, (B, NCP), 1)
    oh = (cls_iota == pred[:, None]).astype(F32)
    prob = jnp.sum(sig * oh[:, :NUM_CLASSES], axis=1)
    prob_ref[0] = prob

    ptrf = ptr_ref[...].astype(F32)
    ptrg = jnp.sum(oh * ptrf, axis=1)

    ii = jax.lax.broadcasted_iota(jnp.int32, (B, B), 0)
    jj = jax.lax.broadcasted_iota(jnp.int32, (B, B), 1)
    eq = pred[:, None] == pred[None, :]
    rank = jnp.sum(jnp.where(eq & (jj < ii), 1.0, 0.0), axis=1)
    pos = ptrg + rank
    pos = pos - 16.0 * jnp.floor(pos * (1.0 / 16.0))
    posi = pos.astype(jnp.int32)
    rows_w = pred * BUF + posi
    rw_ref[0] = rows_w
    idxv = idx_ref[0]
    rr_ref[0] = pred * BUF + idxv

    eqr = rows_w[:, None] == rows_w[None, :]
    dead = jnp.sum(jnp.where(eqr & (jj > ii), 1.0, 0.0), axis=1)
    live = (dead < 0.5)
    live_ref[0] = live.astype(jnp.int32)

    counts = jnp.sum(oh, axis=0)
    pn = ptrf[0] + counts
    pn = pn - 16.0 * jnp.floor(pn * (1.0 / 16.0))
    ptrn_ref[...] = pn[None, :].astype(jnp.int32)

    confp = conf_ref[...]
    cg_rows = jnp.dot(oh, confp, preferred_element_type=F32, precision=jax.lax.Precision.HIGHEST)
    oh_idx = (jax.lax.broadcasted_iota(jnp.int32, (B, BUF), 1) == idxv[:, None]).astype(F32)
    cg_ref[0] = jnp.sum(cg_rows * oh_idx, axis=1)

    ohT = (jax.lax.broadcasted_iota(jnp.int32, (NCP, B), 0) == pred[None, :]).astype(F32)
    oh_pos = (jax.lax.broadcasted_iota(jnp.int32, (B, BUF), 1) == posi[:, None]).astype(F32)
    lf = live.astype(F32)
    vals = jnp.dot(ohT, oh_pos * (lf * prob)[:, None], preferred_element_type=F32, precision=jax.lax.Precision.HIGHEST)
    msk = jnp.dot(ohT, oh_pos * lf[:, None], preferred_element_type=F32, precision=jax.lax.Precision.HIGHEST)
    cn_ref[...] = jnp.where(msk > 0.5, vals, confp)


def _logits_prework(fq, wlin, blin, tgt, ep, idx, ptr, confp):
    i32 = jnp.int32
    outs = pl.pallas_call(
        _logits_body,
        out_shape=[
            jax.ShapeDtypeStruct((B, NUM_CLASSES), F32),
            jax.ShapeDtypeStruct((1, B), i32),
            jax.ShapeDtypeStruct((1, B), F32),
            jax.ShapeDtypeStruct((1, B), i32),
            jax.ShapeDtypeStruct((1, B), i32),
            jax.ShapeDtypeStruct((1, B), i32),
            jax.ShapeDtypeStruct((1, NCP), i32),
            jax.ShapeDtypeStruct((1, B), F32),
            jax.ShapeDtypeStruct((NCP, BUF), F32),
        ],
    )(fq, wlin, blin, tgt, ep, idx, ptr, confp)
    return outs


NB3 = 8
CB3 = DMLP // NB3


def _mlp_body(fq_ref, fk_ref, w1q_ref, w1k_ref, gq_ref, bq_ref, gk_ref, bk_ref,
              w2q_ref, w2k_ref, b2q_ref, b2k_ref, q_ref, k_ref):
    j = pl.program_id(0)
    fqb = fq_ref[...].astype(jnp.bfloat16)
    fkb = fk_ref[...].astype(jnp.bfloat16)
    w1q = w1q_ref[...]
    w1qb = w1q.astype(jnp.bfloat16)
    w1kb = (M * w1k_ref[...] + (1.0 - M) * w1q).astype(jnp.bfloat16)
    hq = jnp.dot(fqb, w1qb, preferred_element_type=F32)
    hk = jnp.dot(fkb, w1kb, preferred_element_type=F32)

    def bn(h, g, b):
        mu = jnp.mean(h, axis=0, keepdims=True)
        var = jnp.mean((h - mu) * (h - mu), axis=0, keepdims=True)
        return g * (h - mu) / jnp.sqrt(var + 1e-5) + b

    gq = gq_ref[...]
    bq = bq_ref[...]
    gk = M * gk_ref[...] + (1.0 - M) * gq
    bk = M * bk_ref[...] + (1.0 - M) * bq
    aq = jnp.maximum(bn(hq, gq, bq), 0.0)
    ak = jnp.maximum(bn(hk, gk, bk), 0.0)

    w2q = w2q_ref[...]
    w2qb = w2q.astype(jnp.bfloat16)
    w2kb = (M * w2k_ref[...] + (1.0 - M) * w2q).astype(jnp.bfloat16)
    cq = jnp.dot(aq.astype(jnp.bfloat16), w2qb, preferred_element_type=F32)
    ck = jnp.dot(ak.astype(jnp.bfloat16), w2kb, preferred_element_type=F32)

    @pl.when(j == 0)
    def _():
        q_ref[...] = cq + b2q_ref[...]
        k_ref[...] = ck + (M * b2k_ref[...] + (1.0 - M) * b2q_ref[...])

    @pl.when(j > 0)
    def _():
        q_ref[...] += cq
        k_ref[...] += ck

    @pl.when(j == NB3 - 1)
    def _():
        def l2n(v):
            n = jnp.sqrt(jnp.sum(v * v, axis=1, keepdims=True))
            return v / jnp.maximum(n, 1e-12)
        q_ref[...] = l2n(q_ref[...])
        k_ref[...] = l2n(k_ref[...])


def _mlp(fq, fk, w1q, w1k, gq, bq, gk, bk, w2q, w2k, b2q, b2k):
    out_sh = jax.ShapeDtypeStruct((B, DIM), F32)
    full = lambda r, c: pl.BlockSpec((r, c), lambda j: (0, 0))
    return pl.pallas_call(
        _mlp_body,
        grid=(NB3,),
        in_specs=[
            full(B, DMLP), full(B, DMLP),
            pl.BlockSpec((DMLP, CB3), lambda j: (0, j)),
            pl.BlockSpec((DMLP, CB3), lambda j: (0, j)),
            pl.BlockSpec((1, CB3), lambda j: (0, j)),
            pl.BlockSpec((1, CB3), lambda j: (0, j)),
            pl.BlockSpec((1, CB3), lambda j: (0, j)),
            pl.BlockSpec((1, CB3), lambda j: (0, j)),
            pl.BlockSpec((CB3, DIM), lambda j: (j, 0)),
            pl.BlockSpec((CB3, DIM), lambda j: (j, 0)),
            full(1, DIM), full(1, DIM),
        ],
        out_specs=[full(B, DIM), full(B, DIM)],
        out_shape=[out_sh, out_sh],
    )(fq, fk, w1q, w1k, gq, bq, gk, bk, w2q, w2k, b2q, b2k)


NB4 = 16
RB4 = (NUM_CLASSES * BUF) // NB4


def _dmem_body(d_ref, k_ref, rw_ref, rr_ref, live_ref, dn_ref, dg_ref):
    r = pl.program_id(0)
    base = r * RB4
    rw = rw_ref[0]
    rr = rr_ref[0]
    lv = live_ref[0]
    kk = k_ref[...]
    d = d_ref[...]

    riota = jax.lax.broadcasted_iota(jnp.int32, (RB4, B), 0) + base
    st = jnp.where((riota == rw[None, :]) & (lv[None, :] > 0), 1.0, 0.0)
    val = jnp.dot(st, kk, preferred_element_type=F32, precision=jax.lax.Precision.HIGHEST)
    msk = jnp.sum(st, axis=1, keepdims=True)
    dn_ref[...] = jnp.where(msk > 0.5, val, d)

    giota = jax.lax.broadcasted_iota(jnp.int32, (B, RB4), 1) + base
    g = jnp.where(giota == rr[:, None], 1.0, 0.0)
    contrib = jnp.dot(g, d, preferred_element_type=F32, precision=jax.lax.Precision.HIGHEST)

    @pl.when(r == 0)
    def _():
        dg_ref[...] = contrib

    @pl.when(r > 0)
    def _():
        dg_ref[...] += contrib


def _dmem(dflat, k, rows_w, rows_r, live):
    return pl.pallas_call(
        _dmem_body,
        grid=(NB4,),
        in_specs=[
            pl.BlockSpec((RB4, DIM), lambda r: (r, 0)),
            pl.BlockSpec((B, DIM), lambda r: (0, 0)),
            pl.BlockSpec((1, B), lambda r: (0, 0)),
            pl.BlockSpec((1, B), lambda r: (0, 0)),
            pl.BlockSpec((1, B), lambda r: (0, 0)),
        ],
        out_specs=[
            pl.BlockSpec((RB4, DIM), lambda r: (r, 0)),
            pl.BlockSpec((B, DIM), lambda r: (0, 0)),
        ],
        out_shape=[
            jax.ShapeDtypeStruct((NUM_CLASSES * BUF, DIM), F32),
            jax.ShapeDtypeStruct((B, DIM), F32),
        ],
    )(dflat, k, rows_w, rows_r, live)


def kernel(im_q, im_k, target, epoch, W_patch_q, Wfc1_q, gamma_q, beta_q, Wfc2_q, bfc2_q,
           W_patch_k, Wfc1_k, gamma_k, beta_k, Wfc2_k, bfc2_k, W_lin, b_lin, conf, d_buf, ptr):
    mq, mk = _img_reduce(im_q, im_k)

    fq, fk = _proj(mq, mk, W_patch_q, W_patch_k)

    idx = jax.random.randint(jax.random.key(42), (B,), 0, BUF)
    ep_row = jnp.full((1, B), epoch, jnp.int32)
    tgt_row = target.astype(jnp.int32).reshape(1, B)
    idx_row = idx.astype(jnp.int32).reshape(1, B)
    ptr_pad = jnp.zeros((1, NCP), jnp.int32).at[0, :NUM_CLASSES].set(ptr)
    conf_pad = jnp.zeros((NCP, BUF), F32).at[:NUM_CLASSES].set(conf)

    (logits, pred, prob, rows_w, rows_r, live, ptr_new_p, conf_g, conf_new_p) = \
        _logits_prework(fq, W_lin, b_lin.reshape(1, NUM_CLASSES), tgt_row, ep_row,
                        idx_row, ptr_pad, conf_pad)

    q, k = _mlp(fq, fk, Wfc1_q, Wfc1_k, gamma_q.reshape(1, DMLP), beta_q.reshape(1, DMLP),
                gamma_k.reshape(1, DMLP), beta_k.reshape(1, DMLP), Wfc2_q, Wfc2_k,
                bfc2_q.reshape(1, DIM), bfc2_k.reshape(1, DIM))

    dflat = d_buf.reshape(NUM_CLASSES * BUF, DIM)
    d_new_flat, d_g = _dmem(dflat, k, rows_w, rows_r, live)

    d_new = d_new_flat.reshape(NUM_CLASSES, BUF, DIM)
    conf_new = conf_new_p[:NUM_CLASSES]
    ptr_new = ptr_new_p[0, :NUM_CLASSES]
    return (q, k, d_g, logits, conf_g[0], d_new, conf_new, ptr_new)


# speedup vs baseline: 3.4287x; 3.4287x over previous
import jax
import jax.numpy as jnp
from jax.experimental import pallas as pl
from jax.experimental.pallas import tpu as pltpu

DIM = 128
M = 0.999
NUM_CLASSES = 1000
BUF = 16
DMLP = 2048
B = 128
NCP = 1024
F32 = jnp.float32


BB1 = 4


def _img_reduce_body(im_ref, r_ref, c_ref, o_ref):
    rm = r_ref[...]
    cm = c_ref[...]
    xb = im_ref[...].reshape(BB1 * 672, 224).astype(jnp.bfloat16)
    z = jnp.dot(rm, xb, preferred_element_type=F32)
    w = jnp.dot(z, cm, preferred_element_type=F32,
                precision=jax.lax.Precision.HIGHEST)
    o_ref[...] = (w * (1.0 / 196.0)).reshape(BB1, 48, 16)


def _img_reduce_one(im3, rsel, csel):
    return pl.pallas_call(
        _img_reduce_body,
        grid=(B // BB1,),
        in_specs=[
            pl.BlockSpec((BB1, 672, 224), lambda b: (b, 0, 0)),
            pl.BlockSpec((48 * BB1, 672 * BB1), lambda b: (0, 0)),
            pl.BlockSpec((224, 16), lambda b: (0, 0)),
        ],
        out_specs=pl.BlockSpec((BB1, 48, 16), lambda b: (b, 0, 0)),
        out_shape=jax.ShapeDtypeStruct((B, 48, 16), F32),
    )(im3, rsel, csel)


def _img_reduce(imq, imk):
    ii = jax.lax.broadcasted_iota(jnp.int32, (48 * BB1, 672 * BB1), 0)
    jj = jax.lax.broadcasted_iota(jnp.int32, (48 * BB1, 672 * BB1), 1)
    rsel = ((ii // 48 == jj // 672) & ((ii % 48) // 16 == (jj % 672) // 224)
            & (ii % 16 == jj % 16)).astype(jnp.bfloat16)
    cj = jax.lax.broadcasted_iota(jnp.int32, (224, 16), 0)
    ck = jax.lax.broadcasted_iota(jnp.int32, (224, 16), 1)
    csel = (cj % 16 == ck).astype(F32)

    oq = _img_reduce_one(imq.reshape(B, 672, 224), rsel, csel)
    ok = _img_reduce_one(imk.reshape(B, 672, 224), rsel, csel)
    return oq.reshape(B, 768), ok.reshape(B, 768)


def _proj_body(mq_ref, mk_ref, wpq_ref, wpk_ref, fq_ref, fk_ref):
    mq = mq_ref[...]
    mk = mk_ref[...]
    wpq = wpq_ref[...]
    wpqb = wpq.astype(jnp.bfloat16).astype(F32)
    wpkb = (M * wpk_ref[...] + (1.0 - M) * wpq).astype(jnp.bfloat16).astype(F32)
    hi = jax.lax.Precision.HIGHEST
    fq_ref[...] = jnp.maximum(jnp.dot(mq, wpqb, preferred_element_type=F32, precision=hi), 0.0)
    fk_ref[...] = jnp.maximum(jnp.dot(mk, wpkb, preferred_element_type=F32, precision=hi), 0.0)


def _proj(mq, mk, wpq, wpk):
    out_sh = jax.ShapeDtypeStruct((B, DMLP), F32)
    return pl.pallas_call(
        _proj_body,
        out_shape=[out_sh, out_sh],
    )(mq, mk, wpq, wpk)


def _logits_body(fq_ref, wlin_ref, blin_ref, tgt_ref, ep_ref, idx_ref, ptr_ref,
                 conf_ref, logits_ref, pred_ref, prob_ref, rw_ref, rr_ref,
                 live_ref, ptrn_ref, cg_ref, cn_ref):
    fqb = fq_ref[...].astype(jnp.bfloat16)
    wlb = wlin_ref[...].astype(jnp.bfloat16)
    lg = jnp.dot(fqb, wlb, preferred_element_type=F32) + blin_ref[...]
    logits_ref[...] = lg

    mx = jnp.max(lg, axis=1, keepdims=True)
    li = jax.lax.broadcasted_iota(jnp.int32, (B, NUM_CLASSES), 1)
    am = jnp.min(jnp.where(lg >= mx, li, NUM_CLASSES), axis=1)
    ep = ep_ref[0]
    tgt = tgt_ref[0]
    pred = jnp.where(ep < 0, tgt, am)
    pred_ref[0] = pred

    sig = 1.0 / (1.0 + jnp.exp(-lg))
    cls_iota = jax.lax.broadcasted_iota(jnp.int32, (B, NCP), 1)
    oh = (cls_iota == pred[:, None]).astype(F32)
    prob = jnp.sum(sig * oh[:, :NUM_CLASSES], axis=1)
    prob_ref[0] = prob

    ptrf = ptr_ref[...].astype(F32)
    ptrg = jnp.sum(oh * ptrf, axis=1)

    ii = jax.lax.broadcasted_iota(jnp.int32, (B, B), 0)
    jj = jax.lax.broadcasted_iota(jnp.int32, (B, B), 1)
    eq = pred[:, None] == pred[None, :]
    rank = jnp.sum(jnp.where(eq & (jj < ii), 1.0, 0.0), axis=1)
    pos = ptrg + rank
    pos = pos - 16.0 * jnp.floor(pos * (1.0 / 16.0))
    posi = pos.astype(jnp.int32)
    rows_w = pred * BUF + posi
    rw_ref[0] = rows_w
    idxv = idx_ref[0]
    rr_ref[0] = pred * BUF + idxv

    eqr = rows_w[:, None] == rows_w[None, :]
    dead = jnp.sum(jnp.where(eqr & (jj > ii), 1.0, 0.0), axis=1)
    live = (dead < 0.5)
    live_ref[0] = live.astype(jnp.int32)

    counts = jnp.sum(oh, axis=0)
    pn = ptrf[0] + counts
    pn = pn - 16.0 * jnp.floor(pn * (1.0 / 16.0))
    ptrn_ref[...] = pn[None, :].astype(jnp.int32)

    confp = conf_ref[...]
    cg_rows = jnp.dot(oh, confp, preferred_element_type=F32, precision=jax.lax.Precision.HIGHEST)
    oh_idx = (jax.lax.broadcasted_iota(jnp.int32, (B, BUF), 1) == idxv[:, None]).astype(F32)
    cg_ref[0] = jnp.sum(cg_rows * oh_idx, axis=1)

    ohT = (jax.lax.broadcasted_iota(jnp.int32, (NCP, B), 0) == pred[None, :]).astype(F32)
    oh_pos = (jax.lax.broadcasted_iota(jnp.int32, (B, BUF), 1) == posi[:, None]).astype(F32)
    lf = live.astype(F32)
    vals = jnp.dot(ohT, oh_pos * (lf * prob)[:, None], preferred_element_type=F32, precision=jax.lax.Precision.HIGHEST)
    msk = jnp.dot(ohT, oh_pos * lf[:, None], preferred_element_type=F32, precision=jax.lax.Precision.HIGHEST)
    cn_ref[...] = jnp.where(msk > 0.5, vals, confp)


def _logits_prework(fq, wlin, blin, tgt, ep, idx, ptr, confp):
    i32 = jnp.int32
    outs = pl.pallas_call(
        _logits_body,
        out_shape=[
            jax.ShapeDtypeStruct((B, NUM_CLASSES), F32),
            jax.ShapeDtypeStruct((1, B), i32),
            jax.ShapeDtypeStruct((1, B), F32),
            jax.ShapeDtypeStruct((1, B), i32),
            jax.ShapeDtypeStruct((1, B), i32),
            jax.ShapeDtypeStruct((1, B), i32),
            jax.ShapeDtypeStruct((1, NCP), i32),
            jax.ShapeDtypeStruct((1, B), F32),
            jax.ShapeDtypeStruct((NCP, BUF), F32),
        ],
    )(fq, wlin, blin, tgt, ep, idx, ptr, confp)
    return outs


NB3 = 8
CB3 = DMLP // NB3


def _mlp_body(fq_ref, fk_ref, w1q_ref, w1k_ref, gq_ref, bq_ref, gk_ref, bk_ref,
              w2q_ref, w2k_ref, b2q_ref, b2k_ref, q_ref, k_ref):
    j = pl.program_id(0)
    fqb = fq_ref[...].astype(jnp.bfloat16)
    fkb = fk_ref[...].astype(jnp.bfloat16)
    w1q = w1q_ref[...]
    w1qb = w1q.astype(jnp.bfloat16)
    w1kb = (M * w1k_ref[...] + (1.0 - M) * w1q).astype(jnp.bfloat16)
    hq = jnp.dot(fqb, w1qb, preferred_element_type=F32)
    hk = jnp.dot(fkb, w1kb, preferred_element_type=F32)

    def bn(h, g, b):
        mu = jnp.mean(h, axis=0, keepdims=True)
        var = jnp.mean((h - mu) * (h - mu), axis=0, keepdims=True)
        return g * (h - mu) / jnp.sqrt(var + 1e-5) + b

    gq = gq_ref[...]
    bq = bq_ref[...]
    gk = M * gk_ref[...] + (1.0 - M) * gq
    bk = M * bk_ref[...] + (1.0 - M) * bq
    aq = jnp.maximum(bn(hq, gq, bq), 0.0)
    ak = jnp.maximum(bn(hk, gk, bk), 0.0)

    w2q = w2q_ref[...]
    w2qb = w2q.astype(jnp.bfloat16)
    w2kb = (M * w2k_ref[...] + (1.0 - M) * w2q).astype(jnp.bfloat16)
    cq = jnp.dot(aq.astype(jnp.bfloat16), w2qb, preferred_element_type=F32)
    ck = jnp.dot(ak.astype(jnp.bfloat16), w2kb, preferred_element_type=F32)

    @pl.when(j == 0)
    def _():
        q_ref[...] = cq + b2q_ref[...]
        k_ref[...] = ck + (M * b2k_ref[...] + (1.0 - M) * b2q_ref[...])

    @pl.when(j > 0)
    def _():
        q_ref[...] += cq
        k_ref[...] += ck

    @pl.when(j == NB3 - 1)
    def _():
        def l2n(v):
            n = jnp.sqrt(jnp.sum(v * v, axis=1, keepdims=True))
            return v / jnp.maximum(n, 1e-12)
        q_ref[...] = l2n(q_ref[...])
        k_ref[...] = l2n(k_ref[...])


def _mlp(fq, fk, w1q, w1k, gq, bq, gk, bk, w2q, w2k, b2q, b2k):
    out_sh = jax.ShapeDtypeStruct((B, DIM), F32)
    full = lambda r, c: pl.BlockSpec((r, c), lambda j: (0, 0))
    return pl.pallas_call(
        _mlp_body,
        grid=(NB3,),
        in_specs=[
            full(B, DMLP), full(B, DMLP),
            pl.BlockSpec((DMLP, CB3), lambda j: (0, j)),
            pl.BlockSpec((DMLP, CB3), lambda j: (0, j)),
            pl.BlockSpec((1, CB3), lambda j: (0, j)),
            pl.BlockSpec((1, CB3), lambda j: (0, j)),
            pl.BlockSpec((1, CB3), lambda j: (0, j)),
            pl.BlockSpec((1, CB3), lambda j: (0, j)),
            pl.BlockSpec((CB3, DIM), lambda j: (j, 0)),
            pl.BlockSpec((CB3, DIM), lambda j: (j, 0)),
            full(1, DIM), full(1, DIM),
        ],
        out_specs=[full(B, DIM), full(B, DIM)],
        out_shape=[out_sh, out_sh],
    )(fq, fk, w1q, w1k, gq, bq, gk, bk, w2q, w2k, b2q, b2k)


NB4 = 16
RB4 = (NUM_CLASSES * BUF) // NB4


def _dmem_body(d_ref, k_ref, rw_ref, rr_ref, live_ref, dn_ref, dg_ref):
    r = pl.program_id(0)
    base = r * RB4
    rw = rw_ref[0]
    rr = rr_ref[0]
    lv = live_ref[0]
    kk = k_ref[...]
    d = d_ref[...]

    riota = jax.lax.broadcasted_iota(jnp.int32, (RB4, B), 0) + base
    st = jnp.where((riota == rw[None, :]) & (lv[None, :] > 0), 1.0, 0.0)
    val = jnp.dot(st, kk, preferred_element_type=F32, precision=jax.lax.Precision.HIGHEST)
    msk = jnp.sum(st, axis=1, keepdims=True)
    dn_ref[...] = jnp.where(msk > 0.5, val, d)

    giota = jax.lax.broadcasted_iota(jnp.int32, (B, RB4), 1) + base
    g = jnp.where(giota == rr[:, None], 1.0, 0.0)
    contrib = jnp.dot(g, d, preferred_element_type=F32, precision=jax.lax.Precision.HIGHEST)

    @pl.when(r == 0)
    def _():
        dg_ref[...] = contrib

    @pl.when(r > 0)
    def _():
        dg_ref[...] += contrib


def _dmem(dflat, k, rows_w, rows_r, live):
    return pl.pallas_call(
        _dmem_body,
        grid=(NB4,),
        in_specs=[
            pl.BlockSpec((RB4, DIM), lambda r: (r, 0)),
            pl.BlockSpec((B, DIM), lambda r: (0, 0)),
            pl.BlockSpec((1, B), lambda r: (0, 0)),
            pl.BlockSpec((1, B), lambda r: (0, 0)),
            pl.BlockSpec((1, B), lambda r: (0, 0)),
        ],
        out_specs=[
            pl.BlockSpec((RB4, DIM), lambda r: (r, 0)),
            pl.BlockSpec((B, DIM), lambda r: (0, 0)),
        ],
        out_shape=[
            jax.ShapeDtypeStruct((NUM_CLASSES * BUF, DIM), F32),
            jax.ShapeDtypeStruct((B, DIM), F32),
        ],
    )(dflat, k, rows_w, rows_r, live)


def kernel(im_q, im_k, target, epoch, W_patch_q, Wfc1_q, gamma_q, beta_q, Wfc2_q, bfc2_q,
           W_patch_k, Wfc1_k, gamma_k, beta_k, Wfc2_k, bfc2_k, W_lin, b_lin, conf, d_buf, ptr):
    mq, mk = _img_reduce(im_q, im_k)

    fq, fk = _proj(mq, mk, W_patch_q, W_patch_k)

    idx = jax.random.randint(jax.random.key(42), (B,), 0, BUF)
    ep_row = jnp.full((1, B), epoch, jnp.int32)
    tgt_row = target.astype(jnp.int32).reshape(1, B)
    idx_row = idx.astype(jnp.int32).reshape(1, B)
    ptr_pad = jnp.zeros((1, NCP), jnp.int32).at[0, :NUM_CLASSES].set(ptr)
    conf_pad = jnp.zeros((NCP, BUF), F32).at[:NUM_CLASSES].set(conf)

    (logits, pred, prob, rows_w, rows_r, live, ptr_new_p, conf_g, conf_new_p) = \
        _logits_prework(fq, W_lin, b_lin.reshape(1, NUM_CLASSES), tgt_row, ep_row,
                        idx_row, ptr_pad, conf_pad)

    q, k = _mlp(fq, fk, Wfc1_q, Wfc1_k, gamma_q.reshape(1, DMLP), beta_q.reshape(1, DMLP),
                gamma_k.reshape(1, DMLP), beta_k.reshape(1, DMLP), Wfc2_q, Wfc2_k,
                bfc2_q.reshape(1, DIM), bfc2_k.reshape(1, DIM))

    dflat = d_buf.reshape(NUM_CLASSES * BUF, DIM)
    d_new_flat, d_g = _dmem(dflat, k, rows_w, rows_r, live)

    d_new = d_new_flat.reshape(NUM_CLASSES, BUF, DIM)
    conf_new = conf_new_p[:NUM_CLASSES]
    ptr_new = ptr_new_p[0, :NUM_CLASSES]
    return (q, k, d_g, logits, conf_g[0], d_new, conf_new, ptr_new)


# speedup vs baseline: 3.4289x; 1.0001x over previous
import jax
import jax.numpy as jnp
from jax.experimental import pallas as pl
from jax.experimental.pallas import tpu as pltpu

DIM = 128
M = 0.999
NUM_CLASSES = 1000
BUF = 16
DMLP = 2048
B = 128
NCP = 1024
F32 = jnp.float32


BB1 = 4


def _img_reduce_body(im_ref, r_ref, c_ref, o_ref):
    rm = r_ref[...]
    cm = c_ref[...]
    xb = im_ref[...].reshape(BB1 * 672, 224).astype(jnp.bfloat16)
    z = jnp.dot(rm, xb, preferred_element_type=F32)
    w = jnp.dot(z, cm, preferred_element_type=F32,
                precision=jax.lax.Precision.HIGHEST)
    o_ref[...] = (w * (1.0 / 196.0)).reshape(BB1, 48, 16)


def _img_reduce_one(im3, rsel, csel):
    return pl.pallas_call(
        _img_reduce_body,
        grid=(B // BB1,),
        in_specs=[
            pl.BlockSpec((BB1, 672, 224), lambda b: (b, 0, 0)),
            pl.BlockSpec((48 * BB1, 672 * BB1), lambda b: (0, 0)),
            pl.BlockSpec((224, 16), lambda b: (0, 0)),
        ],
        out_specs=pl.BlockSpec((BB1, 48, 16), lambda b: (b, 0, 0)),
        out_shape=jax.ShapeDtypeStruct((B, 48, 16), F32),
    )(im3, rsel, csel)


def _img_reduce(imq, imk):
    ii = jax.lax.broadcasted_iota(jnp.int32, (48 * BB1, 672 * BB1), 0)
    jj = jax.lax.broadcasted_iota(jnp.int32, (48 * BB1, 672 * BB1), 1)
    rsel = ((ii // 48 == jj // 672) & ((ii % 48) // 16 == (jj % 672) // 224)
            & (ii % 16 == jj % 16)).astype(jnp.bfloat16)
    cj = jax.lax.broadcasted_iota(jnp.int32, (224, 16), 0)
    ck = jax.lax.broadcasted_iota(jnp.int32, (224, 16), 1)
    csel = (cj % 16 == ck).astype(F32)

    oq = _img_reduce_one(imq.reshape(B, 672, 224), rsel, csel)
    ok = _img_reduce_one(imk.reshape(B, 672, 224), rsel, csel)
    return oq.reshape(B, 768), ok.reshape(B, 768)


def _proj_q_body(mq_ref, wpq_ref, fq_ref):
    hi = jax.lax.Precision.HIGHEST
    wpqb = wpq_ref[...].astype(jnp.bfloat16).astype(F32)
    fq_ref[...] = jnp.maximum(
        jnp.dot(mq_ref[...], wpqb, preferred_element_type=F32, precision=hi), 0.0)


def _proj_k_body(mk_ref, wpq_ref, wpk_ref, fk_ref):
    hi = jax.lax.Precision.HIGHEST
    wpkb = (M * wpk_ref[...] + (1.0 - M) * wpq_ref[...]).astype(jnp.bfloat16).astype(F32)
    fk_ref[...] = jnp.maximum(
        jnp.dot(mk_ref[...], wpkb, preferred_element_type=F32, precision=hi), 0.0)


def _proj_q(mq, wpq):
    return pl.pallas_call(
        _proj_q_body,
        out_shape=jax.ShapeDtypeStruct((B, DMLP), F32),
    )(mq, wpq)


def _proj_k(mk, wpq, wpk):
    return pl.pallas_call(
        _proj_k_body,
        out_shape=jax.ShapeDtypeStruct((B, DMLP), F32),
    )(mk, wpq, wpk)


def _logits_body(fq_ref, wlin_ref, blin_ref, tgt_ref, ep_ref, idx_ref, ptr_ref,
                 conf_ref, logits_ref, pred_ref, prob_ref, rw_ref, rr_ref,
                 live_ref, ptrn_ref, cg_ref, cn_ref):
    fqb = fq_ref[...].astype(jnp.bfloat16)
    wlb = wlin_ref[...].astype(jnp.bfloat16)
    lg = jnp.dot(fqb, wlb, preferred_element_type=F32) + blin_ref[...]
    logits_ref[...] = lg

    mx = jnp.max(lg, axis=1, keepdims=True)
    li = jax.lax.broadcasted_iota(jnp.int32, (B, NUM_CLASSES), 1)
    am = jnp.min(jnp.where(lg >= mx, li, NUM_CLASSES), axis=1)
    ep = ep_ref[0]
    tgt = tgt_ref[0]
    pred = jnp.where(ep < 0, tgt, am)
    pred_ref[0] = pred

    sig = 1.0 / (1.0 + jnp.exp(-lg))
    cls_iota = jax.lax.broadcasted_iota(jnp.int32, (B, NCP), 1)
    oh = (cls_iota == pred[:, None]).astype(F32)
    prob = jnp.sum(sig * oh[:, :NUM_CLASSES], axis=1)
    prob_ref[0] = prob

    ptrf = ptr_ref[...].astype(F32)
    ptrg = jnp.sum(oh * ptrf, axis=1)

    ii = jax.lax.broadcasted_iota(jnp.int32, (B, B), 0)
    jj = jax.lax.broadcasted_iota(jnp.int32, (B, B), 1)
    eq = pred[:, None] == pred[None, :]
    rank = jnp.sum(jnp.where(eq & (jj < ii), 1.0, 0.0), axis=1)
    pos = ptrg + rank
    pos = pos - 16.0 * jnp.floor(pos * (1.0 / 16.0))
    posi = pos.astype(jnp.int32)
    rows_w = pred * BUF + posi
    rw_ref[0] = rows_w
    idxv = idx_ref[0]
    rr_ref[0] = pred * BUF + idxv

    eqr = rows_w[:, None] == rows_w[None, :]
    dead = jnp.sum(jnp.where(eqr & (jj > ii), 1.0, 0.0), axis=1)
    live = (dead < 0.5)
    live_ref[0] = live.astype(jnp.int32)

    counts = jnp.sum(oh, axis=0)
    pn = ptrf[0] + counts
    pn = pn - 16.0 * jnp.floor(pn * (1.0 / 16.0))
    ptrn_ref[...] = pn[None, :].astype(jnp.int32)

    confp = conf_ref[...]
    cg_rows = jnp.dot(oh, confp, preferred_element_type=F32, precision=jax.lax.Precision.HIGHEST)
    oh_idx = (jax.lax.broadcasted_iota(jnp.int32, (B, BUF), 1) == idxv[:, None]).astype(F32)
    cg_ref[0] = jnp.sum(cg_rows * oh_idx, axis=1)

    ohT = (jax.lax.broadcasted_iota(jnp.int32, (NCP, B), 0) == pred[None, :]).astype(F32)
    oh_pos = (jax.lax.broadcasted_iota(jnp.int32, (B, BUF), 1) == posi[:, None]).astype(F32)
    lf = live.astype(F32)
    vals = jnp.dot(ohT, oh_pos * (lf * prob)[:, None], preferred_element_type=F32, precision=jax.lax.Precision.HIGHEST)
    msk = jnp.dot(ohT, oh_pos * lf[:, None], preferred_element_type=F32, precision=jax.lax.Precision.HIGHEST)
    cn_ref[...] = jnp.where(msk > 0.5, vals, confp)


def _logits_prework(fq, wlin, blin, tgt, ep, idx, ptr, confp):
    i32 = jnp.int32
    outs = pl.pallas_call(
        _logits_body,
        out_shape=[
            jax.ShapeDtypeStruct((B, NUM_CLASSES), F32),
            jax.ShapeDtypeStruct((1, B), i32),
            jax.ShapeDtypeStruct((1, B), F32),
            jax.ShapeDtypeStruct((1, B), i32),
            jax.ShapeDtypeStruct((1, B), i32),
            jax.ShapeDtypeStruct((1, B), i32),
            jax.ShapeDtypeStruct((1, NCP), i32),
            jax.ShapeDtypeStruct((1, B), F32),
            jax.ShapeDtypeStruct((NCP, BUF), F32),
        ],
    )(fq, wlin, blin, tgt, ep, idx, ptr, confp)
    return outs


NB3 = 8
CB3 = DMLP // NB3


def _mlp_body(fq_ref, fk_ref, w1q_ref, w1k_ref, gq_ref, bq_ref, gk_ref, bk_ref,
              w2q_ref, w2k_ref, b2q_ref, b2k_ref, q_ref, k_ref):
    j = pl.program_id(0)
    fqb = fq_ref[...].astype(jnp.bfloat16)
    fkb = fk_ref[...].astype(jnp.bfloat16)
    w1q = w1q_ref[...]
    w1qb = w1q.astype(jnp.bfloat16)
    w1kb = (M * w1k_ref[...] + (1.0 - M) * w1q).astype(jnp.bfloat16)
    hq = jnp.dot(fqb, w1qb, preferred_element_type=F32)
    hk = jnp.dot(fkb, w1kb, preferred_element_type=F32)

    def bn(h, g, b):
        mu = jnp.mean(h, axis=0, keepdims=True)
        var = jnp.mean((h - mu) * (h - mu), axis=0, keepdims=True)
        return g * (h - mu) / jnp.sqrt(var + 1e-5) + b

    gq = gq_ref[...]
    bq = bq_ref[...]
    gk = M * gk_ref[...] + (1.0 - M) * gq
    bk = M * bk_ref[...] + (1.0 - M) * bq
    aq = jnp.maximum(bn(hq, gq, bq), 0.0)
    ak = jnp.maximum(bn(hk, gk, bk), 0.0)

    w2q = w2q_ref[...]
    w2qb = w2q.astype(jnp.bfloat16)
    w2kb = (M * w2k_ref[...] + (1.0 - M) * w2q).astype(jnp.bfloat16)
    cq = jnp.dot(aq.astype(jnp.bfloat16), w2qb, preferred_element_type=F32)
    ck = jnp.dot(ak.astype(jnp.bfloat16), w2kb, preferred_element_type=F32)

    @pl.when(j == 0)
    def _():
        q_ref[...] = cq + b2q_ref[...]
        k_ref[...] = ck + (M * b2k_ref[...] + (1.0 - M) * b2q_ref[...])

    @pl.when(j > 0)
    def _():
        q_ref[...] += cq
        k_ref[...] += ck

    @pl.when(j == NB3 - 1)
    def _():
        def l2n(v):
            n = jnp.sqrt(jnp.sum(v * v, axis=1, keepdims=True))
            return v / jnp.maximum(n, 1e-12)
        q_ref[...] = l2n(q_ref[...])
        k_ref[...] = l2n(k_ref[...])


def _mlp(fq, fk, w1q, w1k, gq, bq, gk, bk, w2q, w2k, b2q, b2k):
    out_sh = jax.ShapeDtypeStruct((B, DIM), F32)
    full = lambda r, c: pl.BlockSpec((r, c), lambda j: (0, 0))
    return pl.pallas_call(
        _mlp_body,
        grid=(NB3,),
        in_specs=[
            full(B, DMLP), full(B, DMLP),
            pl.BlockSpec((DMLP, CB3), lambda j: (0, j)),
            pl.BlockSpec((DMLP, CB3), lambda j: (0, j)),
            pl.BlockSpec((1, CB3), lambda j: (0, j)),
            pl.BlockSpec((1, CB3), lambda j: (0, j)),
            pl.BlockSpec((1, CB3), lambda j: (0, j)),
            pl.BlockSpec((1, CB3), lambda j: (0, j)),
            pl.BlockSpec((CB3, DIM), lambda j: (j, 0)),
            pl.BlockSpec((CB3, DIM), lambda j: (j, 0)),
            full(1, DIM), full(1, DIM),
        ],
        out_specs=[full(B, DIM), full(B, DIM)],
        out_shape=[out_sh, out_sh],
    )(fq, fk, w1q, w1k, gq, bq, gk, bk, w2q, w2k, b2q, b2k)


NB4 = 4
RB4 = (NUM_CLASSES * BUF) // NB4


def _dmem_body(d_ref, k_ref, rw_ref, rr_ref, live_ref, dn_ref, dg_ref):
    r = pl.program_id(0)
    base = r * RB4
    rw = rw_ref[0]
    rr = rr_ref[0]
    lv = live_ref[0]
    kk = k_ref[...]
    d = d_ref[...]

    riota = jax.lax.broadcasted_iota(jnp.int32, (RB4, B), 0) + base
    st = jnp.where((riota == rw[None, :]) & (lv[None, :] > 0), 1.0, 0.0)
    val = jnp.dot(st, kk, preferred_element_type=F32, precision=jax.lax.Precision.HIGHEST)
    msk = jnp.sum(st, axis=1, keepdims=True)
    dn_ref[...] = jnp.where(msk > 0.5, val, d)

    giota = jax.lax.broadcasted_iota(jnp.int32, (B, RB4), 1) + base
    g = jnp.where(giota == rr[:, None], 1.0, 0.0)
    contrib = jnp.dot(g, d, preferred_element_type=F32, precision=jax.lax.Precision.HIGHEST)

    @pl.when(r == 0)
    def _():
        dg_ref[...] = contrib

    @pl.when(r > 0)
    def _():
        dg_ref[...] += contrib


def _dmem(dflat, k, rows_w, rows_r, live):
    return pl.pallas_call(
        _dmem_body,
        grid=(NB4,),
        in_specs=[
            pl.BlockSpec((RB4, DIM), lambda r: (r, 0)),
            pl.BlockSpec((B, DIM), lambda r: (0, 0)),
            pl.BlockSpec((1, B), lambda r: (0, 0)),
            pl.BlockSpec((1, B), lambda r: (0, 0)),
            pl.BlockSpec((1, B), lambda r: (0, 0)),
        ],
        out_specs=[
            pl.BlockSpec((RB4, DIM), lambda r: (r, 0)),
            pl.BlockSpec((B, DIM), lambda r: (0, 0)),
        ],
        out_shape=[
            jax.ShapeDtypeStruct((NUM_CLASSES * BUF, DIM), F32),
            jax.ShapeDtypeStruct((B, DIM), F32),
        ],
    )(dflat, k, rows_w, rows_r, live)


def kernel(im_q, im_k, target, epoch, W_patch_q, Wfc1_q, gamma_q, beta_q, Wfc2_q, bfc2_q,
           W_patch_k, Wfc1_k, gamma_k, beta_k, Wfc2_k, bfc2_k, W_lin, b_lin, conf, d_buf, ptr):
    mq, mk = _img_reduce(im_q, im_k)

    fq = _proj_q(mq, W_patch_q)
    fk = _proj_k(mk, W_patch_q, W_patch_k)

    idx = jax.random.randint(jax.random.key(42), (B,), 0, BUF)
    ep_row = jnp.full((1, B), epoch, jnp.int32)
    tgt_row = target.astype(jnp.int32).reshape(1, B)
    idx_row = idx.astype(jnp.int32).reshape(1, B)
    ptr_pad = jnp.zeros((1, NCP), jnp.int32).at[0, :NUM_CLASSES].set(ptr)
    conf_pad = jnp.zeros((NCP, BUF), F32).at[:NUM_CLASSES].set(conf)

    (logits, pred, prob, rows_w, rows_r, live, ptr_new_p, conf_g, conf_new_p) = \
        _logits_prework(fq, W_lin, b_lin.reshape(1, NUM_CLASSES), tgt_row, ep_row,
                        idx_row, ptr_pad, conf_pad)

    q, k = _mlp(fq, fk, Wfc1_q, Wfc1_k, gamma_q.reshape(1, DMLP), beta_q.reshape(1, DMLP),
                gamma_k.reshape(1, DMLP), beta_k.reshape(1, DMLP), Wfc2_q, Wfc2_k,
                bfc2_q.reshape(1, DIM), bfc2_k.reshape(1, DIM))

    dflat = d_buf.reshape(NUM_CLASSES * BUF, DIM)
    d_new_flat, d_g = _dmem(dflat, k, rows_w, rows_r, live)

    d_new = d_new_flat.reshape(NUM_CLASSES, BUF, DIM)
    conf_new = conf_new_p[:NUM_CLASSES]
    ptr_new = ptr_new_p[0, :NUM_CLASSES]
    return (q, k, d_g, logits, conf_g[0], d_new, conf_new, ptr_new)
